# baseline (speedup 1.0000x reference)
import jax
import jax.numpy as jnp
from jax.experimental import pallas as pl
from jax.experimental.pallas import tpu as pltpu

_N = 10000
_D = 128
_K = 10
_BM = 400


def _fused(x_ref, A_ref, W1_ref, b_ref, W2_ref, cl_ref,
           h1_ref, sim_ref, support_ref):
    i = pl.program_id(0)

    @pl.when(i == 0)
    def _prelude():
        x = x_ref[...]
        z = jnp.dot(x, W1_ref[...], preferred_element_type=jnp.float32)
        z = z + b_ref[...]
        support_ref[...] = jnp.dot(z, W2_ref[...],
                                   preferred_element_type=jnp.float32)
        cl = cl_ref[...]
        num = jax.lax.dot_general(z, cl, (((1,), (1,)), ((), ())),
                                  preferred_element_type=jnp.float32)
        z_norm = jnp.sqrt(jnp.sum(z * z, axis=1, keepdims=True))
        c_norm = jnp.sqrt(jnp.sum(cl * cl, axis=1))[None, :]
        sim = num / jnp.maximum(z_norm * c_norm, 1e-8)
        m = jnp.max(sim, axis=1, keepdims=True)
        e = jnp.exp(sim - m)
        sim_ref[...] = e / jnp.sum(e, axis=1, keepdims=True)

    a16 = A_ref[...].astype(jnp.bfloat16)
    s16 = support_ref[...].astype(jnp.bfloat16)
    h1_ref[...] = jnp.tanh(
        jnp.dot(a16, s16, preferred_element_type=jnp.float32))


def kernel(seq1, adj, W_ae1, b_ae1, W_gcn, cluster):
    x = seq1[0]
    A = adj[0]
    b2 = b_ae1.reshape(1, _D)
    grid = (_N // _BM,)
    h1, sim = pl.pallas_call(
        _fused,
        grid=grid,
        in_specs=[
            pl.BlockSpec((_N, _D), lambda i: (0, 0)),
            pl.BlockSpec((_BM, _N), lambda i: (i, 0)),
            pl.BlockSpec((_D, _D), lambda i: (0, 0)),
            pl.BlockSpec((1, _D), lambda i: (0, 0)),
            pl.BlockSpec((_D, _D), lambda i: (0, 0)),
            pl.BlockSpec((_K, _D), lambda i: (0, 0)),
        ],
        out_specs=[
            pl.BlockSpec((_BM, _D), lambda i: (i, 0)),
            pl.BlockSpec((_N, _K), lambda i: (0, 0)),
        ],
        out_shape=[
            jax.ShapeDtypeStruct((_N, _D), jnp.float32),
            jax.ShapeDtypeStruct((_N, _K), jnp.float32),
        ],
        scratch_shapes=[pltpu.VMEM((_N, _D), jnp.float32)],
        compiler_params=pltpu.CompilerParams(
            dimension_semantics=("arbitrary",),
        ),
    )(x, A, W_ae1, b2, W_gcn, cluster)
    return (h1, sim)


# speedup vs baseline: 1.0252x; 1.0252x over previous
import jax
import jax.numpy as jnp
from jax.experimental import pallas as pl
from jax.experimental.pallas import tpu as pltpu

_N = 10000
_D = 128
_K = 10
_BM = 400


def _fused(x_ref, A_ref, W1_ref, b_ref, W2_ref, cl_ref,
           h1_ref, sim_ref, support_ref, z_ref, cln_ref):
    i = pl.program_id(0)

    @pl.when(i == 0)
    def _prelude():
        z = jnp.dot(x_ref[...], W1_ref[...],
                    preferred_element_type=jnp.float32) + b_ref[...]
        z_ref[...] = z
        support_ref[...] = jnp.dot(z, W2_ref[...],
                                   preferred_element_type=jnp.float32)
        cl = cl_ref[...]
        c_norm = jnp.sqrt(jnp.sum(cl * cl, axis=1, keepdims=True))
        cln_ref[...] = cl / jnp.maximum(c_norm, 1e-8)

    h1_ref[...] = jnp.tanh(
        jnp.dot(A_ref[...], support_ref[...],
                preferred_element_type=jnp.float32))

    z_blk = z_ref[pl.ds(i * _BM, _BM), :]
    num = jax.lax.dot_general(z_blk, cln_ref[...], (((1,), (1,)), ((), ())),
                              preferred_element_type=jnp.float32)
    z_norm = jnp.sqrt(jnp.sum(z_blk * z_blk, axis=1, keepdims=True))
    e = jnp.exp(num / jnp.maximum(z_norm, 1e-8))
    sim_ref[...] = e / jnp.sum(e, axis=1, keepdims=True)


def kernel(seq1, adj, W_ae1, b_ae1, W_gcn, cluster):
    x = seq1[0]
    A = adj[0]
    b2 = b_ae1.reshape(1, _D)
    grid = (_N // _BM,)
    h1, sim = pl.pallas_call(
        _fused,
        grid=grid,
        in_specs=[
            pl.BlockSpec((_N, _D), lambda i: (0, 0)),
            pl.BlockSpec((_BM, _N), lambda i: (i, 0)),
            pl.BlockSpec((_D, _D), lambda i: (0, 0)),
            pl.BlockSpec((1, _D), lambda i: (0, 0)),
            pl.BlockSpec((_D, _D), lambda i: (0, 0)),
            pl.BlockSpec((_K, _D), lambda i: (0, 0)),
        ],
        out_specs=[
            pl.BlockSpec((_BM, _D), lambda i: (i, 0)),
            pl.BlockSpec((_BM, _K), lambda i: (i, 0)),
        ],
        out_shape=[
            jax.ShapeDtypeStruct((_N, _D), jnp.float32),
            jax.ShapeDtypeStruct((_N, _K), jnp.float32),
        ],
        scratch_shapes=[
            pltpu.VMEM((_N, _D), jnp.float32),
            pltpu.VMEM((_N, _D), jnp.float32),
            pltpu.VMEM((_K, _D), jnp.float32),
        ],
        compiler_params=pltpu.CompilerParams(
            dimension_semantics=("arbitrary",),
        ),
    )(x, A, W_ae1, b2, W_gcn, cluster)
    return (h1, sim)
